# Initial kernel scaffold; baseline (speedup 1.0000x reference)
import jax
import jax.numpy as jnp
from jax.experimental import pallas as pl


def kernel(x, edge_index, W_l, b_l, W_r):
    raise NotImplementedError("write your pallas kernel here")


# speedup vs baseline: 6.2060x; 6.2060x over previous
import functools

import jax
import jax.numpy as jnp
from jax import lax
from jax.experimental import pallas as pl
from jax.experimental.pallas import tpu as pltpu
from jax.experimental.pallas import tpu_sc as plsc

N_NODES = 10000
N_EDGES = 320000
IN_DIM = 128
OUT_DIM = 126
DEG_COL = 126

NUM_SC = 2
NUM_TILES = 16
NUM_WORKERS = NUM_SC * NUM_TILES
EDGES_PER_W = N_EDGES // NUM_WORKERS
CHUNK = 80
N_CHUNKS = EDGES_PER_W // CHUNK
N_PAD = 10240
ROWS_PER_TILE = N_PAD // NUM_TILES


def _fill_row_ids(zidx_v, base):
    for i in range(CHUNK // 16):
        zidx_v[pl.ds(i * 16, 16)] = lax.iota(jnp.int32, 16) + (base + i * 16)


def _sc_body(y_hbm, src_hbm, dst_hbm, sums_out,
             src_v, dst_v, rows_v, zidx_v, acc_sh, sem):
    c = lax.axis_index("c")
    s = lax.axis_index("s")
    wid = c * NUM_TILES + s
    row0 = s * ROWS_PER_TILE

    def fill_zero(i, carry):
        for j in range(IN_DIM // 16):
            rows_v[i, pl.ds(j * 16, 16)] = jnp.zeros((16,), jnp.float32)
        return carry
    lax.fori_loop(0, CHUNK, fill_zero, 0)

    for j in range(ROWS_PER_TILE // CHUNK):
        _fill_row_ids(zidx_v, row0 + j * CHUNK)
        pltpu.sync_copy(rows_v, acc_sh.at[zidx_v])
    plsc.subcore_barrier()

    def chunk_step(i, carry):
        base = wid * EDGES_PER_W + i * CHUNK
        pltpu.sync_copy(src_hbm.at[pl.ds(base, CHUNK)], src_v)
        pltpu.sync_copy(dst_hbm.at[pl.ds(base, CHUNK)], dst_v)
        pltpu.async_copy(y_hbm.at[src_v], rows_v, sem).wait()
        pltpu.sync_copy(rows_v, acc_sh.at[dst_v], add=True)
        return carry
    lax.fori_loop(0, N_CHUNKS, chunk_step, 0)

    plsc.subcore_barrier()

    for j in range(ROWS_PER_TILE // CHUNK):
        r = row0 + j * CHUNK
        _fill_row_ids(zidx_v, r)
        pltpu.async_copy(acc_sh.at[zidx_v], rows_v, sem).wait()
        pltpu.sync_copy(rows_v, sums_out.at[c, pl.ds(r, CHUNK), :])


_sc_aggregate = functools.partial(
    pl.kernel,
    out_type=jax.ShapeDtypeStruct((NUM_SC, N_PAD, IN_DIM), jnp.float32),
    mesh=plsc.VectorSubcoreMesh(core_axis_name="c", subcore_axis_name="s"),
    scratch_types=[
        pltpu.VMEM((CHUNK,), jnp.int32),
        pltpu.VMEM((CHUNK,), jnp.int32),
        pltpu.VMEM((CHUNK, IN_DIM), jnp.float32),
        pltpu.VMEM((CHUNK,), jnp.int32),
        pltpu.VMEM_SHARED((N_PAD, IN_DIM), jnp.float32),
        pltpu.SemaphoreType.DMA,
    ],
)(_sc_body)


def _tc_proj_body(x_ref, wlp_ref, e_ref, o_ref):
    o_ref[...] = (jnp.dot(x_ref[...], wlp_ref[...],
                          preferred_element_type=jnp.float32,
                          precision=lax.Precision.HIGHEST)
                  + e_ref[...])


def _tc_out_body(parts_ref, x_ref, wr_ref, bl_ref, o_ref):
    a = parts_ref[0] + parts_ref[1]
    deg = jnp.maximum(a[:, DEG_COL:DEG_COL + 1], 1.0)
    mean_l = a[:, :OUT_DIM] / deg
    o = (mean_l
         + jnp.dot(x_ref[...], wr_ref[...], preferred_element_type=jnp.float32,
                   precision=lax.Precision.HIGHEST)
         + bl_ref[...])
    o_ref[...] = jnp.maximum(o, 0.0)


_TC_ROWS = 1000


def _tc_project(x, W_lp, e):
    return pl.pallas_call(
        _tc_proj_body,
        grid=(N_NODES // _TC_ROWS,),
        in_specs=[
            pl.BlockSpec((_TC_ROWS, IN_DIM), lambda i: (i, 0)),
            pl.BlockSpec((IN_DIM, IN_DIM), lambda i: (0, 0)),
            pl.BlockSpec((1, IN_DIM), lambda i: (0, 0)),
        ],
        out_specs=pl.BlockSpec((_TC_ROWS, IN_DIM), lambda i: (i, 0)),
        out_shape=jax.ShapeDtypeStruct((N_NODES, IN_DIM), jnp.float32),
    )(x, W_lp, e)


def _tc_combine(parts, x, W_r, b_l2):
    return pl.pallas_call(
        _tc_out_body,
        grid=(N_NODES // _TC_ROWS,),
        in_specs=[
            pl.BlockSpec((NUM_SC, _TC_ROWS, IN_DIM), lambda i: (0, i, 0)),
            pl.BlockSpec((_TC_ROWS, IN_DIM), lambda i: (i, 0)),
            pl.BlockSpec((IN_DIM, OUT_DIM), lambda i: (0, 0)),
            pl.BlockSpec((1, OUT_DIM), lambda i: (0, 0)),
        ],
        out_specs=pl.BlockSpec((_TC_ROWS, OUT_DIM), lambda i: (i, 0)),
        out_shape=jax.ShapeDtypeStruct((N_NODES, OUT_DIM), jnp.float32),
    )(parts, x, W_r, b_l2)


def kernel(x, edge_index, W_l, b_l, W_r):
    src = edge_index[0].astype(jnp.int32)
    dst = edge_index[1].astype(jnp.int32)
    W_lp = jnp.pad(W_l, ((0, 0), (0, IN_DIM - OUT_DIM)))
    e = jnp.zeros((1, IN_DIM), jnp.float32).at[0, DEG_COL].set(1.0)
    y = _tc_project(x, W_lp, e)
    parts = _sc_aggregate(y, src, dst)
    return _tc_combine(parts, x, W_r, b_l.reshape(1, OUT_DIM))


# speedup vs baseline: 11.1456x; 1.7959x over previous
import functools

import jax
import jax.numpy as jnp
from jax import lax
from jax.experimental import pallas as pl
from jax.experimental.pallas import tpu as pltpu
from jax.experimental.pallas import tpu_sc as plsc

N_NODES = 10000
N_EDGES = 320000
IN_DIM = 128
OUT_DIM = 126
DEG_COL = 126

NUM_SC = 2
NUM_TILES = 16
NUM_WORKERS = NUM_SC * NUM_TILES
EDGES_PER_W = N_EDGES // NUM_WORKERS
CHUNK = 80
N_CHUNKS = EDGES_PER_W // CHUNK
N_PAD = 10240
ROWS_PER_TILE = N_PAD // NUM_TILES


def _fill_row_ids(zidx_v, base):
    for i in range(CHUNK // 16):
        zidx_v[pl.ds(i * 16, 16)] = lax.iota(jnp.int32, 16) + (base + i * 16)


def _sc_body(y_hbm, src_hbm, dst_hbm, sums_out,
             src0, dst0, src1, dst1, rows0, rows1, zidx_v, acc_sh,
             sem_g0, sem_g1, sem_i0, sem_i1):
    c = lax.axis_index("c")
    s = lax.axis_index("s")
    wid = c * NUM_TILES + s
    row0 = s * ROWS_PER_TILE
    ebase = wid * EDGES_PER_W

    def fill_zero(i, carry):
        for j in range(IN_DIM // 16):
            rows0[i, pl.ds(j * 16, 16)] = jnp.zeros((16,), jnp.float32)
        return carry
    lax.fori_loop(0, CHUNK, fill_zero, 0)

    for j in range(ROWS_PER_TILE // CHUNK):
        _fill_row_ids(zidx_v, row0 + j * CHUNK)
        pltpu.sync_copy(rows0, acc_sh.at[zidx_v])
    plsc.subcore_barrier()

    def idx_wait(sv, dv, sem):
        pltpu.make_async_copy(src_hbm.at[pl.ds(0, CHUNK)], sv, sem).wait()
        pltpu.make_async_copy(dst_hbm.at[pl.ds(0, CHUNK)], dv, sem).wait()

    def idx_prefetch(i, sv, dv, sem):
        b = jnp.minimum(ebase + i * CHUNK, N_EDGES - CHUNK)
        pltpu.async_copy(src_hbm.at[pl.ds(b, CHUNK)], sv, sem)
        pltpu.async_copy(dst_hbm.at[pl.ds(b, CHUNK)], dv, sem)

    pltpu.sync_copy(src_hbm.at[pl.ds(ebase, CHUNK)], src0)
    pltpu.sync_copy(dst_hbm.at[pl.ds(ebase, CHUNK)], dst0)
    idx_prefetch(1, src1, dst1, sem_i1)
    pltpu.async_copy(y_hbm.at[src0], rows0, sem_g0)

    def step(i, carry):
        @pl.when(i % 2 == 0)
        def _():
            idx_wait(src1, dst1, sem_i1)
            pltpu.async_copy(y_hbm.at[src1], rows1, sem_g1)
            pltpu.make_async_copy(y_hbm.at[src0], rows0, sem_g0).wait()
            pltpu.sync_copy(rows0, acc_sh.at[dst0], add=True)
            idx_prefetch(i + 2, src0, dst0, sem_i0)

        @pl.when(i % 2 == 1)
        def _():
            idx_wait(src0, dst0, sem_i0)
            pltpu.async_copy(y_hbm.at[src0], rows0, sem_g0)
            pltpu.make_async_copy(y_hbm.at[src1], rows1, sem_g1).wait()
            pltpu.sync_copy(rows1, acc_sh.at[dst1], add=True)
            idx_prefetch(i + 2, src1, dst1, sem_i1)
        return carry
    lax.fori_loop(0, N_CHUNKS - 1, step, 0)

    pltpu.make_async_copy(y_hbm.at[src0], rows0, sem_g0).wait()
    pltpu.sync_copy(rows0, acc_sh.at[dst0], add=True)
    idx_wait(src1, dst1, sem_i1)

    plsc.subcore_barrier()

    for j in range(ROWS_PER_TILE // CHUNK):
        r = row0 + j * CHUNK
        _fill_row_ids(zidx_v, r)
        pltpu.async_copy(acc_sh.at[zidx_v], rows0, sem_g0).wait()
        pltpu.sync_copy(rows0, sums_out.at[c, pl.ds(r, CHUNK), :])


_sc_aggregate = functools.partial(
    pl.kernel,
    out_type=jax.ShapeDtypeStruct((NUM_SC, N_PAD, IN_DIM), jnp.float32),
    mesh=plsc.VectorSubcoreMesh(core_axis_name="c", subcore_axis_name="s"),
    scratch_types=[
        pltpu.VMEM((CHUNK,), jnp.int32),
        pltpu.VMEM((CHUNK,), jnp.int32),
        pltpu.VMEM((CHUNK,), jnp.int32),
        pltpu.VMEM((CHUNK,), jnp.int32),
        pltpu.VMEM((CHUNK, IN_DIM), jnp.float32),
        pltpu.VMEM((CHUNK, IN_DIM), jnp.float32),
        pltpu.VMEM((CHUNK,), jnp.int32),
        pltpu.VMEM_SHARED((N_PAD, IN_DIM), jnp.float32),
        pltpu.SemaphoreType.DMA,
        pltpu.SemaphoreType.DMA,
        pltpu.SemaphoreType.DMA,
        pltpu.SemaphoreType.DMA,
    ],
)(_sc_body)


def _tc_proj_body(x_ref, wlp_ref, e_ref, o_ref):
    o_ref[...] = (jnp.dot(x_ref[...], wlp_ref[...],
                          preferred_element_type=jnp.float32,
                          precision=lax.Precision.HIGHEST)
                  + e_ref[...])


def _tc_out_body(parts_ref, x_ref, wr_ref, bl_ref, o_ref):
    a = parts_ref[0] + parts_ref[1]
    deg = jnp.maximum(a[:, DEG_COL:DEG_COL + 1], 1.0)
    mean_l = a[:, :OUT_DIM] / deg
    o = (mean_l
         + jnp.dot(x_ref[...], wr_ref[...], preferred_element_type=jnp.float32,
                   precision=lax.Precision.HIGHEST)
         + bl_ref[...])
    o_ref[...] = jnp.maximum(o, 0.0)


_TC_ROWS = 1000


def _tc_project(x, W_lp, e):
    return pl.pallas_call(
        _tc_proj_body,
        grid=(N_NODES // _TC_ROWS,),
        in_specs=[
            pl.BlockSpec((_TC_ROWS, IN_DIM), lambda i: (i, 0)),
            pl.BlockSpec((IN_DIM, IN_DIM), lambda i: (0, 0)),
            pl.BlockSpec((1, IN_DIM), lambda i: (0, 0)),
        ],
        out_specs=pl.BlockSpec((_TC_ROWS, IN_DIM), lambda i: (i, 0)),
        out_shape=jax.ShapeDtypeStruct((N_NODES, IN_DIM), jnp.float32),
    )(x, W_lp, e)


def _tc_combine(parts, x, W_r, b_l2):
    return pl.pallas_call(
        _tc_out_body,
        grid=(N_NODES // _TC_ROWS,),
        in_specs=[
            pl.BlockSpec((NUM_SC, _TC_ROWS, IN_DIM), lambda i: (0, i, 0)),
            pl.BlockSpec((_TC_ROWS, IN_DIM), lambda i: (i, 0)),
            pl.BlockSpec((IN_DIM, OUT_DIM), lambda i: (0, 0)),
            pl.BlockSpec((1, OUT_DIM), lambda i: (0, 0)),
        ],
        out_specs=pl.BlockSpec((_TC_ROWS, OUT_DIM), lambda i: (i, 0)),
        out_shape=jax.ShapeDtypeStruct((N_NODES, OUT_DIM), jnp.float32),
    )(parts, x, W_r, b_l2)


def kernel(x, edge_index, W_l, b_l, W_r):
    src = edge_index[0].astype(jnp.int32)
    dst = edge_index[1].astype(jnp.int32)
    W_lp = jnp.pad(W_l, ((0, 0), (0, IN_DIM - OUT_DIM)))
    e = jnp.zeros((1, IN_DIM), jnp.float32).at[0, DEG_COL].set(1.0)
    y = _tc_project(x, W_lp, e)
    parts = _sc_aggregate(y, src, dst)
    return _tc_combine(parts, x, W_r, b_l.reshape(1, OUT_DIM))


# speedup vs baseline: 11.3217x; 1.0158x over previous
import functools

import jax
import jax.numpy as jnp
from jax import lax
from jax.experimental import pallas as pl
from jax.experimental.pallas import tpu as pltpu
from jax.experimental.pallas import tpu_sc as plsc

N_NODES = 10000
N_EDGES = 320000
IN_DIM = 128
OUT_DIM = 126
DEG_COL = 126

NUM_SC = 2
NUM_TILES = 16
NUM_WORKERS = NUM_SC * NUM_TILES
EDGES_PER_W = N_EDGES // NUM_WORKERS
CHUNK = 80
N_CHUNKS = EDGES_PER_W // CHUNK
N_PAD = 10240
ROWS_PER_TILE = N_PAD // NUM_TILES


def _fill_row_ids(zidx_v, base):
    for i in range(CHUNK // 16):
        zidx_v[pl.ds(i * 16, 16)] = lax.iota(jnp.int32, 16) + (base + i * 16)


def _sc_body(y_hbm, src_hbm, dst_hbm, sums_out,
             src0, dst0, src1, dst1, rows0, rows1, zidx_v, acc_sh,
             sem_g0, sem_g1, sem_i0, sem_i1):
    c = lax.axis_index("c")
    s = lax.axis_index("s")
    wid = c * NUM_TILES + s
    row0 = s * ROWS_PER_TILE
    ebase = wid * EDGES_PER_W

    def fill_zero(i, carry):
        for j in range(IN_DIM // 16):
            rows0[i, pl.ds(j * 16, 16)] = jnp.zeros((16,), jnp.float32)
        return carry
    lax.fori_loop(0, CHUNK, fill_zero, 0)

    nz = ROWS_PER_TILE // CHUNK
    for j in range(nz):
        zb = zidx_v if j % 2 == 0 else src1
        if j >= 2:
            pltpu.make_async_copy(rows0, acc_sh.at[zb], sem_i0).wait()
        _fill_row_ids(zb, row0 + j * CHUNK)
        pltpu.async_copy(rows0, acc_sh.at[zb], sem_i0)
    for j in range(min(nz, 2)):
        pltpu.make_async_copy(rows0, acc_sh.at[zidx_v], sem_i0).wait()
    plsc.subcore_barrier()

    def idx_wait(sv, dv, sem):
        pltpu.make_async_copy(src_hbm.at[pl.ds(0, CHUNK)], sv, sem).wait()
        pltpu.make_async_copy(dst_hbm.at[pl.ds(0, CHUNK)], dv, sem).wait()

    def idx_prefetch(i, sv, dv, sem):
        b = jnp.minimum(ebase + i * CHUNK, N_EDGES - CHUNK)
        pltpu.async_copy(src_hbm.at[pl.ds(b, CHUNK)], sv, sem)
        pltpu.async_copy(dst_hbm.at[pl.ds(b, CHUNK)], dv, sem)

    pltpu.sync_copy(src_hbm.at[pl.ds(ebase, CHUNK)], src0)
    pltpu.sync_copy(dst_hbm.at[pl.ds(ebase, CHUNK)], dst0)
    idx_prefetch(1, src1, dst1, sem_i1)
    pltpu.async_copy(y_hbm.at[src0], rows0, sem_g0)

    def step(i, carry):
        @pl.when(i % 2 == 0)
        def _():
            idx_wait(src1, dst1, sem_i1)
            pltpu.async_copy(y_hbm.at[src1], rows1, sem_g1)
            pltpu.make_async_copy(y_hbm.at[src0], rows0, sem_g0).wait()
            pltpu.sync_copy(rows0, acc_sh.at[dst0], add=True)
            idx_prefetch(i + 2, src0, dst0, sem_i0)

        @pl.when(i % 2 == 1)
        def _():
            idx_wait(src0, dst0, sem_i0)
            pltpu.async_copy(y_hbm.at[src0], rows0, sem_g0)
            pltpu.make_async_copy(y_hbm.at[src1], rows1, sem_g1).wait()
            pltpu.sync_copy(rows1, acc_sh.at[dst1], add=True)
            idx_prefetch(i + 2, src1, dst1, sem_i1)
        return carry
    lax.fori_loop(0, N_CHUNKS - 1, step, 0)

    pltpu.make_async_copy(y_hbm.at[src0], rows0, sem_g0).wait()
    pltpu.sync_copy(rows0, acc_sh.at[dst0], add=True)
    idx_wait(src1, dst1, sem_i1)

    plsc.subcore_barrier()

    np_ = ROWS_PER_TILE // CHUNK

    def _bufs(j):
        if j % 2 == 0:
            return zidx_v, rows0, sem_g0, sem_i0
        return src1, rows1, sem_g1, sem_i1

    for j in range(np_ + 1):
        if j < np_:
            zb, rb, sg, si = _bufs(j)
            if j >= 2:
                pltpu.make_async_copy(
                    rb, sums_out.at[c, pl.ds(row0 + (j - 2) * CHUNK, CHUNK), :],
                    si).wait()
            _fill_row_ids(zb, row0 + j * CHUNK)
            pltpu.async_copy(acc_sh.at[zb], rb, sg)
        if j >= 1:
            zb, rb, sg, si = _bufs(j - 1)
            pltpu.make_async_copy(acc_sh.at[zb], rb, sg).wait()
            pltpu.async_copy(
                rb, sums_out.at[c, pl.ds(row0 + (j - 1) * CHUNK, CHUNK), :], si)
    for jp in (np_ - 2, np_ - 1):
        zb, rb, sg, si = _bufs(jp)
        pltpu.make_async_copy(
            rb, sums_out.at[c, pl.ds(row0 + jp * CHUNK, CHUNK), :], si).wait()


_sc_aggregate = functools.partial(
    pl.kernel,
    out_type=jax.ShapeDtypeStruct((NUM_SC, N_PAD, IN_DIM), jnp.float32),
    mesh=plsc.VectorSubcoreMesh(core_axis_name="c", subcore_axis_name="s"),
    scratch_types=[
        pltpu.VMEM((CHUNK,), jnp.int32),
        pltpu.VMEM((CHUNK,), jnp.int32),
        pltpu.VMEM((CHUNK,), jnp.int32),
        pltpu.VMEM((CHUNK,), jnp.int32),
        pltpu.VMEM((CHUNK, IN_DIM), jnp.float32),
        pltpu.VMEM((CHUNK, IN_DIM), jnp.float32),
        pltpu.VMEM((CHUNK,), jnp.int32),
        pltpu.VMEM_SHARED((N_PAD, IN_DIM), jnp.float32),
        pltpu.SemaphoreType.DMA,
        pltpu.SemaphoreType.DMA,
        pltpu.SemaphoreType.DMA,
        pltpu.SemaphoreType.DMA,
    ],
)(_sc_body)


def _tc_proj_body(x_ref, wlp_ref, e_ref, o_ref):
    o_ref[...] = (jnp.dot(x_ref[...], wlp_ref[...],
                          preferred_element_type=jnp.float32,
                          precision=lax.Precision.HIGHEST)
                  + e_ref[...])


def _tc_out_body(parts_ref, x_ref, wr_ref, bl_ref, o_ref):
    a = parts_ref[0] + parts_ref[1]
    deg = jnp.maximum(a[:, DEG_COL:DEG_COL + 1], 1.0)
    mean_l = a[:, :OUT_DIM] / deg
    o = (mean_l
         + jnp.dot(x_ref[...], wr_ref[...], preferred_element_type=jnp.float32,
                   precision=lax.Precision.HIGHEST)
         + bl_ref[...])
    o_ref[...] = jnp.maximum(o, 0.0)


_TC_ROWS = 1000


def _tc_project(x, W_lp, e):
    return pl.pallas_call(
        _tc_proj_body,
        grid=(N_NODES // _TC_ROWS,),
        in_specs=[
            pl.BlockSpec((_TC_ROWS, IN_DIM), lambda i: (i, 0)),
            pl.BlockSpec((IN_DIM, IN_DIM), lambda i: (0, 0)),
            pl.BlockSpec((1, IN_DIM), lambda i: (0, 0)),
        ],
        out_specs=pl.BlockSpec((_TC_ROWS, IN_DIM), lambda i: (i, 0)),
        out_shape=jax.ShapeDtypeStruct((N_NODES, IN_DIM), jnp.float32),
    )(x, W_lp, e)


def _tc_combine(parts, x, W_r, b_l2):
    return pl.pallas_call(
        _tc_out_body,
        grid=(N_NODES // _TC_ROWS,),
        in_specs=[
            pl.BlockSpec((NUM_SC, _TC_ROWS, IN_DIM), lambda i: (0, i, 0)),
            pl.BlockSpec((_TC_ROWS, IN_DIM), lambda i: (i, 0)),
            pl.BlockSpec((IN_DIM, OUT_DIM), lambda i: (0, 0)),
            pl.BlockSpec((1, OUT_DIM), lambda i: (0, 0)),
        ],
        out_specs=pl.BlockSpec((_TC_ROWS, OUT_DIM), lambda i: (i, 0)),
        out_shape=jax.ShapeDtypeStruct((N_NODES, OUT_DIM), jnp.float32),
    )(parts, x, W_r, b_l2)


def kernel(x, edge_index, W_l, b_l, W_r):
    src = edge_index[0].astype(jnp.int32)
    dst = edge_index[1].astype(jnp.int32)
    W_lp = jnp.pad(W_l, ((0, 0), (0, IN_DIM - OUT_DIM)))
    e = jnp.zeros((1, IN_DIM), jnp.float32).at[0, DEG_COL].set(1.0)
    y = _tc_project(x, W_lp, e)
    parts = _sc_aggregate(y, src, dst)
    return _tc_combine(parts, x, W_r, b_l.reshape(1, OUT_DIM))
